# Initial kernel scaffold; baseline (speedup 1.0000x reference)
import jax
import jax.numpy as jnp
from jax.experimental import pallas as pl


def kernel(query, response, memory_keys, memory_values, negs, W_in, W_out, W_lin):
    raise NotImplementedError("write your pallas kernel here")


# speedup vs baseline: 14.2077x; 14.2077x over previous
import functools

import jax
import jax.numpy as jnp
from jax import lax
from jax.experimental import pallas as pl
from jax.experimental.pallas import tpu as pltpu
from jax.experimental.pallas import tpu_sc as plsc

NW = 32
CH = 32
L = 20
D = 32
IDX_MINOR = 128


def _renorm_body(w_ref, o_ref):
    x = w_ref[...]
    n = jnp.sqrt(jnp.sum(x * x, axis=1, keepdims=True))
    scale = jnp.where(n > 10.0, 10.0 / (n + 1e-7), 1.0)
    o_ref[...] = x * scale


def _renorm(w):
    v, d = w.shape
    bs = 5000
    return pl.pallas_call(
        _renorm_body,
        grid=(v // bs,),
        in_specs=[pl.BlockSpec((bs, d), lambda i: (i, 0))],
        out_specs=pl.BlockSpec((bs, d), lambda i: (i, 0)),
        out_shape=jax.ShapeDtypeStruct((v, d), jnp.float32),
    )(w)


def _gather_mean(table_in, table_out, idx_in_flat, idx_out_flat, s_in, s_out):
    mesh = plsc.VectorSubcoreMesh(core_axis_name="c", subcore_axis_name="s")
    segs_in_w = s_in // NW
    segs_out_w = s_out // NW
    ngather = CH * L // IDX_MINOR

    @functools.partial(
        pl.kernel,
        mesh=mesh,
        compiler_params=pltpu.CompilerParams(use_tc_tiling_on_sc=False),
        out_type=[
            jax.ShapeDtypeStruct((s_in, D), jnp.float32),
            jax.ShapeDtypeStruct((s_out, D), jnp.float32),
        ],
        scratch_types=[
            pltpu.VMEM((CH * L,), jnp.int32),
            pltpu.VMEM((CH * L, D), jnp.float32),
            pltpu.VMEM((CH, D), jnp.float32),
            pltpu.SemaphoreType.DMA,
        ],
    )
    def k(tin, tout, iin, iout, pooled_in, pooled_out, idxv, rowsv, outv, sem):
        wid = lax.axis_index("s") * 2 + lax.axis_index("c")

        def do_table(table, idx_flat, pooled, segs_w):
            seg0 = wid * segs_w

            def chunk_body(c, carry):
                seg_base = seg0 + c * CH
                ibase = seg_base * L
                pltpu.sync_copy(idx_flat.at[pl.ds(ibase, CH * L)], idxv)
                handles = []
                for j in range(ngather):
                    handles.append(
                        pltpu.async_copy(
                            table.at[idxv.at[pl.ds(j * IDX_MINOR, IDX_MINOR)]],
                            rowsv.at[pl.ds(j * IDX_MINOR, IDX_MINOR), :],
                            sem,
                        )
                    )
                for h in handles:
                    h.wait()

                def seg_body(s, carry2):
                    base = s * L
                    acc0 = rowsv[base, pl.ds(0, 16)]
                    acc1 = rowsv[base, pl.ds(16, 16)]
                    for l in range(1, L):
                        acc0 = acc0 + rowsv[base + l, pl.ds(0, 16)]
                        acc1 = acc1 + rowsv[base + l, pl.ds(16, 16)]
                    outv[s, pl.ds(0, 16)] = acc0 * (1.0 / L)
                    outv[s, pl.ds(16, 16)] = acc1 * (1.0 / L)
                    return carry2

                lax.fori_loop(0, CH, seg_body, 0)
                pltpu.sync_copy(outv, pooled.at[pl.ds(seg_base, CH), :])
                return carry

            lax.fori_loop(0, segs_w // CH, chunk_body, 0)

        do_table(tin, iin, pooled_in, segs_in_w)
        do_table(tout, iout, pooled_out, segs_out_w)

    return k(table_in, table_out, idx_in_flat, idx_out_flat)


def _dense_body(q_ref, mk_ref, mv_ref, negs_ref, resp_ref, wl_ref,
                xe_ref, ne_ref):
    q = q_ref[...]
    mk = mk_ref[...]
    mv = mv_ref[...]
    negs = negs_ref[...]
    resp = resp_ref[...]
    wl = wl_ref[...]

    num = jnp.sum(q[:, None, :] * mk, axis=2)
    qn = jnp.sqrt(jnp.sum(q * q, axis=1))
    mkn = jnp.sqrt(jnp.sum(mk * mk, axis=2))
    den = jnp.maximum(qn, 1e-8)[:, None] * jnp.maximum(mkn, 1e-8)
    sim = num / den
    sm = jax.nn.softmax(sim, axis=1)
    vr = jnp.sum(sm[:, :, None] * mv, axis=1)
    res = jnp.dot(vr, wl.T, preferred_element_type=jnp.float32)
    xe_ref[...] = jnp.broadcast_to(res[:, None, :], negs.shape)
    midx = lax.broadcasted_iota(jnp.int32, negs.shape, 1)
    ne_ref[...] = jnp.where(midx == 0, resp[:, None, :], negs)


def _dense(q, mk, mv, negs_p, resp, w_lin):
    b, m, d = mk.shape
    n = negs_p.shape[1]
    bs = 128
    grid = (b // bs,)
    return pl.pallas_call(
        _dense_body,
        grid=grid,
        in_specs=[
            pl.BlockSpec((bs, d), lambda i: (i, 0)),
            pl.BlockSpec((bs, m, d), lambda i: (i, 0, 0)),
            pl.BlockSpec((bs, m, d), lambda i: (i, 0, 0)),
            pl.BlockSpec((bs, n, d), lambda i: (i, 0, 0)),
            pl.BlockSpec((bs, d), lambda i: (i, 0)),
            pl.BlockSpec((d, d), lambda i: (0, 0)),
        ],
        out_specs=[
            pl.BlockSpec((bs, n, d), lambda i: (i, 0, 0)),
            pl.BlockSpec((bs, n, d), lambda i: (i, 0, 0)),
        ],
        out_shape=[
            jax.ShapeDtypeStruct((b, n, d), jnp.float32),
            jax.ShapeDtypeStruct((b, n, d), jnp.float32),
        ],
    )(q, mk, mv, negs_p, resp, w_lin)


def kernel(query, response, memory_keys, memory_values, negs, W_in, W_out,
           W_lin):
    b, l = query.shape
    m = memory_keys.shape[1]
    n = negs.shape[1]

    wn_in = _renorm(W_in)
    wn_out = _renorm(W_out)

    idx_in = jnp.concatenate(
        [query.reshape(-1), memory_keys.reshape(-1), memory_values.reshape(-1)]
    )
    idx_out = jnp.concatenate([response.reshape(-1), negs.reshape(-1)])

    s_in = b + 2 * b * m
    s_out = b + b * n
    pooled_in, pooled_out = _gather_mean(wn_in, wn_out, idx_in, idx_out,
                                         s_in, s_out)

    q = pooled_in[:b]
    mk = pooled_in[b:b + b * m].reshape(b, m, D)
    mv = pooled_in[b + b * m:].reshape(b, m, D)
    resp = pooled_out[:b]
    negs_p = pooled_out[b:].reshape(b, n, D)

    return _dense(q, mk, mv, negs_p, resp, W_lin)


# speedup vs baseline: 19.9323x; 1.4029x over previous
import functools

import jax
import jax.numpy as jnp
from jax import lax
from jax.experimental import pallas as pl
from jax.experimental.pallas import tpu as pltpu
from jax.experimental.pallas import tpu_sc as plsc

NW = 32
CH = 32
L = 20
D = 32
IDX_MINOR = 128
CHL = CH * L
NG = CHL // IDX_MINOR


def _renorm_body(w_ref, o_ref):
    x = w_ref[...]
    n = jnp.sqrt(jnp.sum(x * x, axis=1, keepdims=True))
    scale = jnp.where(n > 10.0, 10.0 / (n + 1e-7), 1.0)
    o_ref[...] = x * scale


def _renorm(w):
    v, d = w.shape
    bs = 5000
    return pl.pallas_call(
        _renorm_body,
        grid=(v // bs,),
        in_specs=[pl.BlockSpec((bs, d), lambda i: (i, 0))],
        out_specs=pl.BlockSpec((bs, d), lambda i: (i, 0)),
        out_shape=jax.ShapeDtypeStruct((v, d), jnp.float32),
    )(w)


def _gather_mean(table_in, table_out, mk_idx, mv_idx, q_idx, negs_idx,
                 resp_idx, s_in, s_out):
    mesh = plsc.VectorSubcoreMesh(core_axis_name="c", subcore_axis_name="s")

    @functools.partial(
        pl.kernel,
        mesh=mesh,
        compiler_params=pltpu.CompilerParams(use_tc_tiling_on_sc=False),
        out_type=[
            jax.ShapeDtypeStruct((s_in, D), jnp.float32),
            jax.ShapeDtypeStruct((s_out, D), jnp.float32),
        ],
        scratch_types=[
            pltpu.VMEM((2 * CHL,), jnp.int32),
            pltpu.VMEM((2 * CHL, D), jnp.float32),
            pltpu.VMEM((2 * CH, D), jnp.float32),
            pltpu.SemaphoreType.DMA,
            pltpu.SemaphoreType.DMA,
        ],
    )
    def k(tin, tout, mki, mvi, qi, ngi, rsi, pooled_in, pooled_out,
          idxv, rowsv, outv, sem0, sem1):
        wid = lax.axis_index("s") * 2 + lax.axis_index("c")
        sems = (sem0, sem1)

        def gather_descs(table, p, sem):
            return [
                (
                    table.at[idxv.at[pl.ds(p * CHL + j * IDX_MINOR,
                                           IDX_MINOR)]],
                    rowsv.at[pl.ds(p * CHL + j * IDX_MINOR, IDX_MINOR), :],
                    sem,
                )
                for j in range(NG)
            ]

        def prefetch(table, idx_ref, seg0, c, p):
            pltpu.sync_copy(
                idx_ref.at[pl.ds((seg0 + c * CH) * L, CHL)],
                idxv.at[pl.ds(p * CHL, CHL)],
            )
            for src, dst, sem in gather_descs(table, p, sems[p]):
                pltpu.async_copy(src, dst, sem)

        def process(table, idx_ref, pooled, row0, seg0, c, p):
            for src, dst, sem in gather_descs(table, p, sems[p]):
                pltpu.make_async_copy(src, dst, sem).wait()

            def seg_body(s, carry):
                base = p * CHL + s * L
                acc0 = rowsv[base, pl.ds(0, 16)]
                acc1 = rowsv[base, pl.ds(16, 16)]
                for l in range(1, L):
                    acc0 = acc0 + rowsv[base + l, pl.ds(0, 16)]
                    acc1 = acc1 + rowsv[base + l, pl.ds(16, 16)]
                outv[p * CH + s, pl.ds(0, 16)] = acc0 * (1.0 / L)
                outv[p * CH + s, pl.ds(16, 16)] = acc1 * (1.0 / L)
                return carry

            lax.fori_loop(0, CH, seg_body, 0)
            pltpu.sync_copy(
                outv.at[pl.ds(p * CH, CH), :],
                pooled.at[pl.ds(row0 + seg0 + c * CH, CH), :],
            )

        def do_range(table, idx_ref, pooled, row0, segs_w):
            seg0 = wid * segs_w
            n = segs_w // CH
            if n == 1:
                prefetch(table, idx_ref, seg0, 0, 0)
                process(table, idx_ref, pooled, row0, seg0, 0, 0)
                return
            assert n % 2 == 0 and n >= 4
            prefetch(table, idx_ref, seg0, 0, 0)
            prefetch(table, idx_ref, seg0, 1, 1)

            def pair_body(k2, carry):
                c = 2 * k2
                process(table, idx_ref, pooled, row0, seg0, c, 0)
                prefetch(table, idx_ref, seg0, c + 2, 0)
                process(table, idx_ref, pooled, row0, seg0, c + 1, 1)
                prefetch(table, idx_ref, seg0, c + 3, 1)
                return carry

            lax.fori_loop(0, n // 2 - 1, pair_body, 0)
            process(table, idx_ref, pooled, row0, seg0, n - 2, 0)
            process(table, idx_ref, pooled, row0, seg0, n - 1, 1)

        n_mk = mki.shape[0] // L
        n_mv = mvi.shape[0] // L
        n_q = qi.shape[0] // L
        n_ng = ngi.shape[0] // L
        do_range(tin, mki, pooled_in, 0, n_mk // NW)
        do_range(tin, mvi, pooled_in, n_mk, n_mv // NW)
        do_range(tin, qi, pooled_in, n_mk + n_mv, n_q // NW)
        do_range(tout, ngi, pooled_out, 0, n_ng // NW)
        do_range(tout, rsi, pooled_out, n_ng, n_q // NW)

    return k(table_in, table_out, mk_idx, mv_idx, q_idx, negs_idx, resp_idx)


def _dense_body(mk_ref, mv_ref, q_ref, negs_ref, resp_ref, wl_ref,
                xe_ref, ne_ref):
    bs = q_ref.shape[0]
    m = mk_ref.shape[0] // bs
    n = negs_ref.shape[0] // bs
    q = q_ref[...]
    mk = mk_ref[...].reshape(bs, m, D)
    mv = mv_ref[...].reshape(bs, m, D)
    negs = negs_ref[...].reshape(bs, n, D)
    resp = resp_ref[...]
    wl = wl_ref[...]

    num = jnp.sum(q[:, None, :] * mk, axis=2)
    qn = jnp.sqrt(jnp.sum(q * q, axis=1))
    mkn = jnp.sqrt(jnp.sum(mk * mk, axis=2))
    den = jnp.maximum(qn, 1e-8)[:, None] * jnp.maximum(mkn, 1e-8)
    sim = num / den
    sm = jax.nn.softmax(sim, axis=1)
    vr = jnp.sum(sm[:, :, None] * mv, axis=1)
    res = jnp.dot(vr, wl.T, preferred_element_type=jnp.float32)
    xe_ref[...] = jnp.broadcast_to(res[:, None, :], (bs, n, D))
    midx = lax.broadcasted_iota(jnp.int32, (bs, n, D), 1)
    ne_ref[...] = jnp.where(midx == 0, resp[:, None, :], negs)


def _dense(pooled_in, pooled_out, w_lin, b, m, n):
    bs = 128
    grid = (b // bs,)
    mk_blocks = b * m // (bs * m)
    q_block0 = 2 * b * m // bs
    resp_block0 = b * n // bs
    return pl.pallas_call(
        _dense_body,
        grid=grid,
        in_specs=[
            pl.BlockSpec((bs * m, D), lambda i: (i, 0)),
            pl.BlockSpec((bs * m, D), lambda i: (i + mk_blocks, 0)),
            pl.BlockSpec((bs, D), lambda i: (i + q_block0, 0)),
            pl.BlockSpec((bs * n, D), lambda i: (i, 0)),
            pl.BlockSpec((bs, D), lambda i: (i + resp_block0, 0)),
            pl.BlockSpec((D, D), lambda i: (0, 0)),
        ],
        out_specs=[
            pl.BlockSpec((bs, n, D), lambda i: (i, 0, 0)),
            pl.BlockSpec((bs, n, D), lambda i: (i, 0, 0)),
        ],
        out_shape=[
            jax.ShapeDtypeStruct((b, n, D), jnp.float32),
            jax.ShapeDtypeStruct((b, n, D), jnp.float32),
        ],
    )(pooled_in, pooled_in, pooled_in, pooled_out, pooled_out, w_lin)


def kernel(query, response, memory_keys, memory_values, negs, W_in, W_out,
           W_lin):
    b, l = query.shape
    m = memory_keys.shape[1]
    n = negs.shape[1]

    wn_in = _renorm(W_in)
    wn_out = _renorm(W_out)

    s_in = 2 * b * m + b
    s_out = b * n + b
    pooled_in, pooled_out = _gather_mean(
        wn_in, wn_out,
        memory_keys.reshape(-1), memory_values.reshape(-1),
        query.reshape(-1), negs.reshape(-1), response.reshape(-1),
        s_in, s_out,
    )

    return _dense(pooled_in, pooled_out, W_lin, b, m, n)


# speedup vs baseline: 23.7400x; 1.1910x over previous
import functools

import jax
import jax.numpy as jnp
from jax import lax
from jax.experimental import pallas as pl
from jax.experimental.pallas import tpu as pltpu
from jax.experimental.pallas import tpu_sc as plsc

NW = 32
CH = 32
L = 20
D = 32
IDX_MINOR = 128
CHL = CH * L
NG = CHL // IDX_MINOR


def _renorm_body(a_ref, b_ref, oa_ref, ob_ref):
    gi = lax.broadcasted_iota(jnp.int32, (128, 128), 0) >> 5
    gj = lax.broadcasted_iota(jnp.int32, (128, 128), 1) >> 5
    g = jnp.where(gi == gj, 1.0, 0.0).astype(jnp.float32)

    def renorm(x):
        n2 = jnp.dot(x * x, g, preferred_element_type=jnp.float32)
        n = jnp.sqrt(n2)
        scale = jnp.where(n > 10.0, 10.0 / (n + 1e-7), 1.0)
        return x * scale

    oa_ref[...] = renorm(a_ref[...])
    ob_ref[...] = renorm(b_ref[...])


def _renorm2(w_a, w_b):
    v4 = w_a.shape[0]
    bs = 5000
    spec = pl.BlockSpec((bs, 128), lambda i: (i, 0))
    return pl.pallas_call(
        _renorm_body,
        grid=(v4 // bs,),
        in_specs=[spec, spec],
        out_specs=[spec, spec],
        out_shape=[
            jax.ShapeDtypeStruct((v4, 128), jnp.float32),
            jax.ShapeDtypeStruct((v4, 128), jnp.float32),
        ],
    )(w_a, w_b)


def _gather_mean(table_in, table_out, mk_idx, mv_idx, q_idx, negs_idx,
                 resp_idx, s_in, s_out):
    mesh = plsc.VectorSubcoreMesh(core_axis_name="c", subcore_axis_name="s")

    @functools.partial(
        pl.kernel,
        mesh=mesh,
        compiler_params=pltpu.CompilerParams(use_tc_tiling_on_sc=False),
        out_type=[
            jax.ShapeDtypeStruct((s_in, D), jnp.float32),
            jax.ShapeDtypeStruct((s_out, D), jnp.float32),
        ],
        scratch_types=[
            pltpu.VMEM((2 * CHL,), jnp.int32),
            pltpu.VMEM((2 * CHL, D), jnp.float32),
            pltpu.VMEM((2 * CH, D), jnp.float32),
            pltpu.SemaphoreType.DMA,
            pltpu.SemaphoreType.DMA,
        ],
    )
    def k(tin, tout, mki, mvi, qi, ngi, rsi, pooled_in, pooled_out,
          idxv, rowsv, outv, sem0, sem1):
        wid = lax.axis_index("s") * 2 + lax.axis_index("c")
        sems = (sem0, sem1)

        def gather_descs(table, p, sem):
            return [
                (
                    table.at[idxv.at[pl.ds(p * CHL + j * IDX_MINOR,
                                           IDX_MINOR)]],
                    rowsv.at[pl.ds(p * CHL + j * IDX_MINOR, IDX_MINOR), :],
                    sem,
                )
                for j in range(NG)
            ]

        def prefetch(table, idx_ref, seg0, c, p):
            pltpu.sync_copy(
                idx_ref.at[pl.ds((seg0 + c * CH) * L, CHL)],
                idxv.at[pl.ds(p * CHL, CHL)],
            )
            for src, dst, sem in gather_descs(table, p, sems[p]):
                pltpu.async_copy(src, dst, sem)

        def process(table, idx_ref, pooled, row0, seg0, c, p):
            for src, dst, sem in gather_descs(table, p, sems[p]):
                pltpu.make_async_copy(src, dst, sem).wait()

            def seg_body(s, carry):
                base = p * CHL + s * L
                acc0 = rowsv[base, pl.ds(0, 16)]
                acc1 = rowsv[base, pl.ds(16, 16)]
                for l in range(1, L):
                    acc0 = acc0 + rowsv[base + l, pl.ds(0, 16)]
                    acc1 = acc1 + rowsv[base + l, pl.ds(16, 16)]
                outv[p * CH + s, pl.ds(0, 16)] = acc0 * (1.0 / L)
                outv[p * CH + s, pl.ds(16, 16)] = acc1 * (1.0 / L)
                return carry

            lax.fori_loop(0, CH, seg_body, 0)
            pltpu.sync_copy(
                outv.at[pl.ds(p * CH, CH), :],
                pooled.at[pl.ds(row0 + seg0 + c * CH, CH), :],
            )

        def do_range(table, idx_ref, pooled, row0, segs_w):
            seg0 = wid * segs_w
            n = segs_w // CH
            if n == 1:
                prefetch(table, idx_ref, seg0, 0, 0)
                process(table, idx_ref, pooled, row0, seg0, 0, 0)
                return
            assert n % 2 == 0 and n >= 4
            prefetch(table, idx_ref, seg0, 0, 0)
            prefetch(table, idx_ref, seg0, 1, 1)

            def pair_body(k2, carry):
                c = 2 * k2
                process(table, idx_ref, pooled, row0, seg0, c, 0)
                prefetch(table, idx_ref, seg0, c + 2, 0)
                process(table, idx_ref, pooled, row0, seg0, c + 1, 1)
                prefetch(table, idx_ref, seg0, c + 3, 1)
                return carry

            lax.fori_loop(0, n // 2 - 1, pair_body, 0)
            process(table, idx_ref, pooled, row0, seg0, n - 2, 0)
            process(table, idx_ref, pooled, row0, seg0, n - 1, 1)

        n_mk = mki.shape[0] // L
        n_mv = mvi.shape[0] // L
        n_q = qi.shape[0] // L
        n_ng = ngi.shape[0] // L
        do_range(tin, mki, pooled_in, 0, n_mk // NW)
        do_range(tin, mvi, pooled_in, n_mk, n_mv // NW)
        do_range(tin, qi, pooled_in, n_mk + n_mv, n_q // NW)
        do_range(tout, ngi, pooled_out, 0, n_ng // NW)
        do_range(tout, rsi, pooled_out, n_ng, n_q // NW)

    return k(table_in, table_out, mk_idx, mv_idx, q_idx, negs_idx, resp_idx)


def _dense_body(mk_ref, mv_ref, q_ref, negs_ref, resp_ref, wl_ref,
                xe_ref, ne_ref):
    bs = q_ref.shape[0]
    m = mk_ref.shape[0] // bs
    n = negs_ref.shape[0] // bs
    q = q_ref[...]
    mk = mk_ref[...].reshape(bs, m, D)
    mv = mv_ref[...].reshape(bs, m, D)
    negs = negs_ref[...].reshape(bs, n, D)
    resp = resp_ref[...]
    wl = wl_ref[...]

    num = jnp.sum(q[:, None, :] * mk, axis=2)
    qn = jnp.sqrt(jnp.sum(q * q, axis=1))
    mkn = jnp.sqrt(jnp.sum(mk * mk, axis=2))
    den = jnp.maximum(qn, 1e-8)[:, None] * jnp.maximum(mkn, 1e-8)
    sim = num / den
    sm = jax.nn.softmax(sim, axis=1)
    vr = jnp.sum(sm[:, :, None] * mv, axis=1)
    res = jnp.dot(vr, wl.T, preferred_element_type=jnp.float32)
    xe_ref[...] = jnp.broadcast_to(res[:, None, :], (bs, n, D))
    midx = lax.broadcasted_iota(jnp.int32, (bs, n, D), 1)
    ne_ref[...] = jnp.where(midx == 0, resp[:, None, :], negs)


def _dense(pooled_in, pooled_out, w_lin, b, m, n):
    bs = 128
    grid = (b // bs,)
    mk_blocks = b * m // (bs * m)
    q_block0 = 2 * b * m // bs
    resp_block0 = b * n // bs
    return pl.pallas_call(
        _dense_body,
        grid=grid,
        in_specs=[
            pl.BlockSpec((bs * m, D), lambda i: (i, 0)),
            pl.BlockSpec((bs * m, D), lambda i: (i + mk_blocks, 0)),
            pl.BlockSpec((bs, D), lambda i: (i + q_block0, 0)),
            pl.BlockSpec((bs * n, D), lambda i: (i, 0)),
            pl.BlockSpec((bs, D), lambda i: (i + resp_block0, 0)),
            pl.BlockSpec((D, D), lambda i: (0, 0)),
        ],
        out_specs=[
            pl.BlockSpec((bs, n, D), lambda i: (i, 0, 0)),
            pl.BlockSpec((bs, n, D), lambda i: (i, 0, 0)),
        ],
        out_shape=[
            jax.ShapeDtypeStruct((b, n, D), jnp.float32),
            jax.ShapeDtypeStruct((b, n, D), jnp.float32),
        ],
    )(pooled_in, pooled_in, pooled_in, pooled_out, pooled_out, w_lin)


def kernel(query, response, memory_keys, memory_values, negs, W_in, W_out,
           W_lin):
    b, l = query.shape
    m = memory_keys.shape[1]
    n = negs.shape[1]

    v = W_in.shape[0]
    wn_in128, wn_out128 = _renorm2(W_in.reshape(v // 4, 4 * D),
                                   W_out.reshape(v // 4, 4 * D))
    wn_in = wn_in128.reshape(v, D)
    wn_out = wn_out128.reshape(v, D)

    s_in = 2 * b * m + b
    s_out = b * n + b
    pooled_in, pooled_out = _gather_mean(
        wn_in, wn_out,
        memory_keys.reshape(-1), memory_values.reshape(-1),
        query.reshape(-1), negs.reshape(-1), response.reshape(-1),
        s_in, s_out,
    )

    return _dense(pooled_in, pooled_out, W_lin, b, m, n)


# speedup vs baseline: 26.9835x; 1.1366x over previous
import functools

import jax
import jax.numpy as jnp
from jax import lax
from jax.experimental import pallas as pl
from jax.experimental.pallas import tpu as pltpu
from jax.experimental.pallas import tpu_sc as plsc

NW = 32
CH = 32
L = 20
D = 32
IDX_MINOR = 128
CHL = CH * L
NG = CHL // IDX_MINOR


def _renorm_body(a_ref, b_ref, oa_ref, ob_ref):
    gi = lax.broadcasted_iota(jnp.int32, (128, 128), 0) >> 5
    gj = lax.broadcasted_iota(jnp.int32, (128, 128), 1) >> 5
    g = jnp.where(gi == gj, 1.0, 0.0).astype(jnp.float32)

    def renorm(x):
        n2 = jnp.dot(x * x, g, preferred_element_type=jnp.float32)
        n = jnp.sqrt(n2)
        scale = jnp.where(n > 10.0, 10.0 / (n + 1e-7), 1.0)
        return x * scale

    oa_ref[...] = renorm(a_ref[...])
    ob_ref[...] = renorm(b_ref[...])


def _renorm2(w_a, w_b):
    v4 = w_a.shape[0]
    bs = 5000
    spec = pl.BlockSpec((bs, 128), lambda i: (i, 0))
    return pl.pallas_call(
        _renorm_body,
        grid=(v4 // bs,),
        in_specs=[spec, spec],
        out_specs=[spec, spec],
        out_shape=[
            jax.ShapeDtypeStruct((v4, 128), jnp.float32),
            jax.ShapeDtypeStruct((v4, 128), jnp.float32),
        ],
    )(w_a, w_b)


MAXSEG = 50


def _windows(total):
    out = []
    off = 0
    while total - off > 128:
        out.append((off, 128))
        off += 128
    out.append((off, total - off))
    return out


def _gather_mean(table_in, table_out, mk_idx, mv_idx, q_idx, negs_idx,
                 resp_idx, s_in, s_out):
    mesh = plsc.VectorSubcoreMesh(core_axis_name="c", subcore_axis_name="s")

    @functools.partial(
        pl.kernel,
        mesh=mesh,
        compiler_params=pltpu.CompilerParams(use_tc_tiling_on_sc=False),
        out_type=[
            jax.ShapeDtypeStruct((s_in, D), jnp.float32),
            jax.ShapeDtypeStruct((s_out, D), jnp.float32),
        ],
        scratch_types=[
            pltpu.VMEM((MAXSEG * L,), jnp.int32),
            pltpu.VMEM((MAXSEG * L,), jnp.int32),
            pltpu.VMEM((MAXSEG * L, D), jnp.float32),
            pltpu.VMEM((MAXSEG * L, D), jnp.float32),
            pltpu.VMEM((MAXSEG, D), jnp.float32),
            pltpu.VMEM((20, D), jnp.float32),
            pltpu.VMEM((32, D), jnp.float32),
            pltpu.SemaphoreType.DMA,
            pltpu.SemaphoreType.DMA,
        ],
    )
    def k(tin, tout, mki, mvi, qi, ngi, rsi, pooled_in, pooled_out,
          idx0, idx1, rows0, rows1, out50, out20, out32, sem0, sem1):
        wid = lax.axis_index("s") * 2 + lax.axis_index("c")
        bufs = ((idx0, rows0, sem0), (idx1, rows1, sem1))

        def gather_descs(table, p, nidx):
            idxv, rowsv, sem = bufs[p]
            return [
                (
                    table.at[idxv.at[pl.ds(off, sz)]],
                    rowsv.at[pl.ds(off, sz), :],
                    sem,
                )
                for off, sz in _windows(nidx)
            ]

        def prefetch(table, idx_row, p, nidx):
            pltpu.sync_copy(idx_row, bufs[p][0].at[pl.ds(0, nidx)])
            for src, dst, sem in gather_descs(table, p, nidx):
                pltpu.async_copy(src, dst, sem)

        def reduce_store(rowsv, outv, segs):
            def seg_body(s, carry):
                base = s * L
                acc0 = rowsv[base, pl.ds(0, 16)]
                acc1 = rowsv[base, pl.ds(16, 16)]
                for l in range(1, L):
                    acc0 = acc0 + rowsv[base + l, pl.ds(0, 16)]
                    acc1 = acc1 + rowsv[base + l, pl.ds(16, 16)]
                outv[s, pl.ds(0, 16)] = acc0 * (1.0 / L)
                outv[s, pl.ds(16, 16)] = acc1 * (1.0 / L)
                return carry

            lax.fori_loop(0, segs, seg_body, 0)

        def process(table, pooled, outv, row0, p, segs):
            for src, dst, sem in gather_descs(table, p, segs * L):
                pltpu.make_async_copy(src, dst, sem).wait()
            reduce_store(bufs[p][1], outv, segs)
            pltpu.sync_copy(outv, pooled.at[pl.ds(row0, segs), :])

        def do_batch_stream(table, idx2d, pooled, outv, row0, segs):
            b0 = wid * 32

            def pre(c, p):
                prefetch(table, idx2d.at[b0 + c], p, segs * L)

            def proc(c, p):
                process(table, pooled, outv, row0 + (b0 + c) * segs, p,
                        segs)

            pre(0, 0)
            pre(1, 1)

            def pair_body(k2, carry):
                c = 2 * k2
                proc(c, 0)
                pre(c + 2, 0)
                proc(c + 1, 1)
                pre(c + 3, 1)
                return carry

            lax.fori_loop(0, 15, pair_body, 0)
            proc(30, 0)
            proc(31, 1)

        def do_flat_stream(table, idx_flat, pooled, row0):
            segs = 32
            b0 = wid * segs
            prefetch(table, idx_flat.at[pl.ds(b0 * L, segs * L)], 0,
                     segs * L)
            process(table, pooled, out32, row0 + b0, 0, segs)

        b = mki.shape[0]
        m = mki.shape[1] // L
        n = ngi.shape[1] // L
        do_batch_stream(tin, mki, pooled_in, out50, 0, m)
        do_batch_stream(tin, mvi, pooled_in, out50, b * m, m)
        do_flat_stream(tin, qi, pooled_in, 2 * b * m)
        do_batch_stream(tout, ngi, pooled_out, out20, 0, n)
        do_flat_stream(tout, rsi, pooled_out, b * n)

    return k(table_in, table_out, mk_idx, mv_idx, q_idx, negs_idx, resp_idx)


def _dense_body(mk_ref, mv_ref, q_ref, negs_ref, resp_ref, wl_ref,
                xe_ref, ne_ref):
    bs = q_ref.shape[0]
    m = mk_ref.shape[0] // bs
    n = negs_ref.shape[0] // bs
    q = q_ref[...]
    mk = mk_ref[...].reshape(bs, m, D)
    mv = mv_ref[...].reshape(bs, m, D)
    negs = negs_ref[...].reshape(bs, n, D)
    resp = resp_ref[...]
    wl = wl_ref[...]

    num = jnp.sum(q[:, None, :] * mk, axis=2)
    qn = jnp.sqrt(jnp.sum(q * q, axis=1))
    mkn = jnp.sqrt(jnp.sum(mk * mk, axis=2))
    den = jnp.maximum(qn, 1e-8)[:, None] * jnp.maximum(mkn, 1e-8)
    sim = num / den
    sm = jax.nn.softmax(sim, axis=1)
    vr = jnp.sum(sm[:, :, None] * mv, axis=1)
    res = jnp.dot(vr, wl.T, preferred_element_type=jnp.float32)
    xe_ref[...] = jnp.broadcast_to(res[:, None, :], (bs, n, D))
    midx = lax.broadcasted_iota(jnp.int32, (bs, n, D), 1)
    ne_ref[...] = jnp.where(midx == 0, resp[:, None, :], negs)


def _dense(pooled_in, pooled_out, w_lin, b, m, n):
    bs = 128
    grid = (b // bs,)
    mk_blocks = b * m // (bs * m)
    q_block0 = 2 * b * m // bs
    resp_block0 = b * n // bs
    return pl.pallas_call(
        _dense_body,
        grid=grid,
        in_specs=[
            pl.BlockSpec((bs * m, D), lambda i: (i, 0)),
            pl.BlockSpec((bs * m, D), lambda i: (i + mk_blocks, 0)),
            pl.BlockSpec((bs, D), lambda i: (i + q_block0, 0)),
            pl.BlockSpec((bs * n, D), lambda i: (i, 0)),
            pl.BlockSpec((bs, D), lambda i: (i + resp_block0, 0)),
            pl.BlockSpec((D, D), lambda i: (0, 0)),
        ],
        out_specs=[
            pl.BlockSpec((bs, n, D), lambda i: (i, 0, 0)),
            pl.BlockSpec((bs, n, D), lambda i: (i, 0, 0)),
        ],
        out_shape=[
            jax.ShapeDtypeStruct((b, n, D), jnp.float32),
            jax.ShapeDtypeStruct((b, n, D), jnp.float32),
        ],
    )(pooled_in, pooled_in, pooled_in, pooled_out, pooled_out, w_lin)


def kernel(query, response, memory_keys, memory_values, negs, W_in, W_out,
           W_lin):
    b, l = query.shape
    m = memory_keys.shape[1]
    n = negs.shape[1]

    v = W_in.shape[0]
    wn_in128, wn_out128 = _renorm2(W_in.reshape(v // 4, 4 * D),
                                   W_out.reshape(v // 4, 4 * D))
    wn_in = wn_in128.reshape(v, D)
    wn_out = wn_out128.reshape(v, D)

    s_in = 2 * b * m + b
    s_out = b * n + b
    pooled_in, pooled_out = _gather_mean(
        wn_in, wn_out,
        memory_keys.reshape(b, m * l), memory_values.reshape(b, m * l),
        query.reshape(-1), negs.reshape(b, n * l), response.reshape(-1),
        s_in, s_out,
    )

    return _dense(pooled_in, pooled_out, W_lin, b, m, n)


# speedup vs baseline: 27.0294x; 1.0017x over previous
import functools

import jax
import jax.numpy as jnp
from jax import lax
from jax.experimental import pallas as pl
from jax.experimental.pallas import tpu as pltpu
from jax.experimental.pallas import tpu_sc as plsc

NW = 32
CH = 32
L = 20
D = 32
IDX_MINOR = 128
CHL = CH * L
NG = CHL // IDX_MINOR


def _renorm_body(a_ref, b_ref, oa_ref, ob_ref):
    gi = lax.broadcasted_iota(jnp.int32, (128, 128), 0) >> 5
    gj = lax.broadcasted_iota(jnp.int32, (128, 128), 1) >> 5
    g = jnp.where(gi == gj, 1.0, 0.0).astype(jnp.float32)

    def renorm(x):
        n2 = jnp.dot(x * x, g, preferred_element_type=jnp.float32)
        n = jnp.sqrt(n2)
        scale = jnp.where(n > 10.0, 10.0 / (n + 1e-7), 1.0)
        return x * scale

    oa_ref[...] = renorm(a_ref[...])
    ob_ref[...] = renorm(b_ref[...])


def _renorm2(w_a, w_b):
    v4 = w_a.shape[0]
    bs = 5000
    spec = pl.BlockSpec((bs, 128), lambda i: (i, 0))
    return pl.pallas_call(
        _renorm_body,
        grid=(v4 // bs,),
        in_specs=[spec, spec],
        out_specs=[spec, spec],
        out_shape=[
            jax.ShapeDtypeStruct((v4, 128), jnp.float32),
            jax.ShapeDtypeStruct((v4, 128), jnp.float32),
        ],
    )(w_a, w_b)


MAXSEG = 50


def _windows(total):
    out = []
    off = 0
    while total - off > 128:
        out.append((off, 128))
        off += 128
    out.append((off, total - off))
    return out


def _gather_mean(table_in, table_out, mk_idx, mv_idx, q_idx, negs_idx,
                 resp_idx, s_in, s_out):
    mesh = plsc.VectorSubcoreMesh(core_axis_name="c", subcore_axis_name="s")

    @functools.partial(
        pl.kernel,
        mesh=mesh,
        compiler_params=pltpu.CompilerParams(use_tc_tiling_on_sc=False),
        out_type=[
            jax.ShapeDtypeStruct((s_in, D), jnp.float32),
            jax.ShapeDtypeStruct((s_out, D), jnp.float32),
        ],
        scratch_types=[
            pltpu.VMEM((MAXSEG * L,), jnp.int32),
            pltpu.VMEM((MAXSEG * L,), jnp.int32),
            pltpu.VMEM((MAXSEG * L,), jnp.int32),
            pltpu.VMEM((MAXSEG * L, D), jnp.float32),
            pltpu.VMEM((MAXSEG * L, D), jnp.float32),
            pltpu.VMEM((MAXSEG * L, D), jnp.float32),
            pltpu.VMEM((MAXSEG, D), jnp.float32),
            pltpu.VMEM((20, D), jnp.float32),
            pltpu.VMEM((32, D), jnp.float32),
            pltpu.SemaphoreType.DMA,
            pltpu.SemaphoreType.DMA,
            pltpu.SemaphoreType.DMA,
        ],
    )
    def k(tin, tout, mki, mvi, qi, ngi, rsi, pooled_in, pooled_out,
          idx0, idx1, idx2, rows0, rows1, rows2, out50, out20, out32,
          sem0, sem1, sem2):
        wid = lax.axis_index("s") * 2 + lax.axis_index("c")
        bufs = ((idx0, rows0, sem0), (idx1, rows1, sem1),
                (idx2, rows2, sem2))

        def gather_descs(table, p, nidx):
            idxv, rowsv, sem = bufs[p]
            return [
                (
                    table.at[idxv.at[pl.ds(off, sz)]],
                    rowsv.at[pl.ds(off, sz), :],
                    sem,
                )
                for off, sz in _windows(nidx)
            ]

        def prefetch(table, idx_row, p, nidx):
            pltpu.sync_copy(idx_row, bufs[p][0].at[pl.ds(0, nidx)])
            for src, dst, sem in gather_descs(table, p, nidx):
                pltpu.async_copy(src, dst, sem)

        def reduce_store(rowsv, outv, segs):
            def seg_body(s, carry):
                base = s * L
                acc0 = rowsv[base, pl.ds(0, 16)]
                acc1 = rowsv[base, pl.ds(16, 16)]
                for l in range(1, L):
                    acc0 = acc0 + rowsv[base + l, pl.ds(0, 16)]
                    acc1 = acc1 + rowsv[base + l, pl.ds(16, 16)]
                outv[s, pl.ds(0, 16)] = acc0 * (1.0 / L)
                outv[s, pl.ds(16, 16)] = acc1 * (1.0 / L)
                return carry

            lax.fori_loop(0, segs, seg_body, 0)

        def process(table, pooled, outv, row0, p, segs):
            for src, dst, sem in gather_descs(table, p, segs * L):
                pltpu.make_async_copy(src, dst, sem).wait()
            reduce_store(bufs[p][1], outv, segs)
            pltpu.sync_copy(outv, pooled.at[pl.ds(row0, segs), :])

        def do_batch_stream(table, idx2d, pooled, outv, row0, segs):
            b0 = wid * 32

            def pre(c, p):
                prefetch(table, idx2d.at[b0 + c], p, segs * L)

            def proc(c, p):
                process(table, pooled, outv, row0 + (b0 + c) * segs, p,
                        segs)

            pre(0, 0)
            pre(1, 1)
            pre(2, 2)

            def ring_body(k3, carry):
                c = 3 * k3
                proc(c, 0)
                pre(c + 3, 0)
                proc(c + 1, 1)
                pre(c + 4, 1)
                proc(c + 2, 2)
                pre(c + 5, 2)
                return carry

            lax.fori_loop(0, 9, ring_body, 0)
            proc(27, 0)
            pre(30, 0)
            proc(28, 1)
            pre(31, 1)
            proc(29, 2)
            proc(30, 0)
            proc(31, 1)

        def do_flat_stream(table, idx_flat, pooled, row0):
            segs = 32
            b0 = wid * segs
            prefetch(table, idx_flat.at[pl.ds(b0 * L, segs * L)], 0,
                     segs * L)
            process(table, pooled, out32, row0 + b0, 0, segs)

        b = mki.shape[0]
        m = mki.shape[1] // L
        n = ngi.shape[1] // L
        do_batch_stream(tin, mki, pooled_in, out50, 0, m)
        do_batch_stream(tin, mvi, pooled_in, out50, b * m, m)
        do_flat_stream(tin, qi, pooled_in, 2 * b * m)
        do_batch_stream(tout, ngi, pooled_out, out20, 0, n)
        do_flat_stream(tout, rsi, pooled_out, b * n)

    return k(table_in, table_out, mk_idx, mv_idx, q_idx, negs_idx, resp_idx)


def _dense_body(mk_ref, mv_ref, q_ref, negs_ref, resp_ref, wl_ref,
                xe_ref, ne_ref):
    bs = q_ref.shape[0]
    m = mk_ref.shape[0] // bs
    n = negs_ref.shape[0] // bs
    q = q_ref[...]
    mk = mk_ref[...].reshape(bs, m, D)
    mv = mv_ref[...].reshape(bs, m, D)
    negs = negs_ref[...].reshape(bs, n, D)
    resp = resp_ref[...]
    wl = wl_ref[...]

    num = jnp.sum(q[:, None, :] * mk, axis=2)
    qn = jnp.sqrt(jnp.sum(q * q, axis=1))
    mkn = jnp.sqrt(jnp.sum(mk * mk, axis=2))
    den = jnp.maximum(qn, 1e-8)[:, None] * jnp.maximum(mkn, 1e-8)
    sim = num / den
    sm = jax.nn.softmax(sim, axis=1)
    vr = jnp.sum(sm[:, :, None] * mv, axis=1)
    res = jnp.dot(vr, wl.T, preferred_element_type=jnp.float32)
    xe_ref[...] = jnp.broadcast_to(res[:, None, :], (bs, n, D))
    midx = lax.broadcasted_iota(jnp.int32, (bs, n, D), 1)
    ne_ref[...] = jnp.where(midx == 0, resp[:, None, :], negs)


def _dense(pooled_in, pooled_out, w_lin, b, m, n):
    bs = 128
    grid = (b // bs,)
    mk_blocks = b * m // (bs * m)
    q_block0 = 2 * b * m // bs
    resp_block0 = b * n // bs
    return pl.pallas_call(
        _dense_body,
        grid=grid,
        in_specs=[
            pl.BlockSpec((bs * m, D), lambda i: (i, 0)),
            pl.BlockSpec((bs * m, D), lambda i: (i + mk_blocks, 0)),
            pl.BlockSpec((bs, D), lambda i: (i + q_block0, 0)),
            pl.BlockSpec((bs * n, D), lambda i: (i, 0)),
            pl.BlockSpec((bs, D), lambda i: (i + resp_block0, 0)),
            pl.BlockSpec((D, D), lambda i: (0, 0)),
        ],
        out_specs=[
            pl.BlockSpec((bs, n, D), lambda i: (i, 0, 0)),
            pl.BlockSpec((bs, n, D), lambda i: (i, 0, 0)),
        ],
        out_shape=[
            jax.ShapeDtypeStruct((b, n, D), jnp.float32),
            jax.ShapeDtypeStruct((b, n, D), jnp.float32),
        ],
    )(pooled_in, pooled_in, pooled_in, pooled_out, pooled_out, w_lin)


def kernel(query, response, memory_keys, memory_values, negs, W_in, W_out,
           W_lin):
    b, l = query.shape
    m = memory_keys.shape[1]
    n = negs.shape[1]

    v = W_in.shape[0]
    wn_in128, wn_out128 = _renorm2(W_in.reshape(v // 4, 4 * D),
                                   W_out.reshape(v // 4, 4 * D))
    wn_in = wn_in128.reshape(v, D)
    wn_out = wn_out128.reshape(v, D)

    s_in = 2 * b * m + b
    s_out = b * n + b
    pooled_in, pooled_out = _gather_mean(
        wn_in, wn_out,
        memory_keys.reshape(b, m * l), memory_values.reshape(b, m * l),
        query.reshape(-1), negs.reshape(b, n * l), response.reshape(-1),
        s_in, s_out,
    )

    return _dense(pooled_in, pooled_out, W_lin, b, m, n)
